# baseline (speedup 1.0000x reference)
import functools

import jax
import jax.numpy as jnp
from jax import lax
from jax.experimental import pallas as pl
from jax.experimental.pallas import tpu as pltpu
from jax.experimental.pallas import tpu_sc as plsc

SIDE = 128
GRID_SZ = SIDE * SIDE * SIDE
KVOL = 27
C = 16
NC = 2
NS = 16
NW = NC * NS
CHUNK = 128
NCHUNK = 25
VR = CHUNK * NCHUNK
NPAD = NW * VR
ZROW = NPAD - 1


def _widx():
    return lax.axis_index("s") * NC + lax.axis_index("c")


def _rulebook_body(z_hbm, y_hbm, x_hbm, grid_hbm, nidx_hbm,
                   zb, yb, xb, linb, mb, gb, nb, sem):
    tb = _widx() * VR

    def k_loop(k, _):
        dz = k // 9 - 1
        dy = (k // 3) % 3 - 1
        dx = k % 3 - 1

        def c_loop(ci, _):
            base = tb + ci * CHUNK
            pltpu.async_copy(z_hbm.at[pl.ds(base, CHUNK)], zb, sem).wait()
            pltpu.async_copy(y_hbm.at[pl.ds(base, CHUNK)], yb, sem).wait()
            pltpu.async_copy(x_hbm.at[pl.ds(base, CHUNK)], xb, sem).wait()
            for g in range(CHUNK // 16):
                sl = pl.ds(g * 16, 16)
                zz = zb[sl] + dz
                yy = yb[sl] + dy
                xx = xb[sl] + dx
                valid = ((zz >= 0) & (zz < SIDE) & (yy >= 0) & (yy < SIDE)
                         & (xx >= 0) & (xx < SIDE))
                lin = (zz * SIDE + yy) * SIDE + xx
                linb[sl] = jnp.where(valid, lin, 0)
                mb[sl] = jnp.where(valid, 1, 0)
            pltpu.async_copy(grid_hbm.at[linb], gb, sem).wait()
            for g in range(CHUNK // 16):
                sl = pl.ds(g * 16, 16)
                gv = gb[sl]
                ok = (mb[sl] == 1) & (gv >= 0)
                nb[sl] = jnp.where(ok, gv, ZROW)
            pltpu.async_copy(nb, nidx_hbm.at[pl.ds(k * NPAD + base, CHUNK)],
                             sem).wait()
            return 0

        lax.fori_loop(0, NCHUNK, c_loop, 0)
        return 0

    lax.fori_loop(0, KVOL, k_loop, 0)


def _rulebook(zf, yf, xf, grid):
    mesh = plsc.VectorSubcoreMesh(core_axis_name="c", subcore_axis_name="s")
    f = pl.kernel(
        _rulebook_body,
        mesh=mesh,
        compiler_params=pltpu.CompilerParams(use_tc_tiling_on_sc=False),
        out_type=jax.ShapeDtypeStruct((KVOL * NPAD,), jnp.int32),
        scratch_types=[
            pltpu.VMEM((CHUNK,), jnp.int32),
            pltpu.VMEM((CHUNK,), jnp.int32),
            pltpu.VMEM((CHUNK,), jnp.int32),
            pltpu.VMEM((CHUNK,), jnp.int32),
            pltpu.VMEM((CHUNK,), jnp.int32),
            pltpu.VMEM((CHUNK,), jnp.int32),
            pltpu.VMEM((CHUNK,), jnp.int32),
            pltpu.SemaphoreType.DMA,
        ],
    )
    return f(zf, yf, xf, grid)


def _transpose16(r):
    iot = lax.iota(jnp.int32, 16)
    for b in (1, 2, 4, 8):
        maskb = (iot & b) != 0
        pxor = iot ^ b
        nr = list(r)
        for i in range(16):
            if i & b:
                continue
            j = i | b
            lo, hi = r[i], r[j]
            hip = hi.at[pxor].get(mode="promise_in_bounds")
            lop = lo.at[pxor].get(mode="promise_in_bounds")
            nr[i] = jnp.where(maskb, hip, lo)
            nr[j] = jnp.where(maskb, hi, lop)
        r = nr
    return r


def _conv_body(table_hbm, nidx_hbm, w_hbm, out_hbm,
               acc, gbuf, nixb, obuf, wsh, wsm, sem):
    tb = _widx() * VR
    zero = jnp.zeros((16,), jnp.float32)

    @pl.when(lax.axis_index("s") == 0)
    def _stage_w():
        pltpu.async_copy(w_hbm, wsh, sem).wait()

    plsc.subcore_barrier()

    def z_loop(i, _):
        for j in range(8):
            acc[pl.ds((i * 8 + j) * 16, 16)] = zero
        return 0

    lax.fori_loop(0, (C * VR) // (16 * 8), z_loop, 0)

    def k_loop(k, _):
        pltpu.async_copy(wsh.at[k], wsm, sem).wait()
        pltpu.async_copy(nidx_hbm.at[pl.ds(k * NPAD + tb, VR)], nixb,
                         sem).wait()

        def c_loop(ci, _):
            pltpu.async_copy(table_hbm.at[nixb.at[pl.ds(ci * CHUNK, CHUNK)]],
                             gbuf, sem).wait()
            ws = [[wsm[c, d] for d in range(C)] for c in range(C)]
            for g in range(CHUNK // 16):
                rows = [gbuf[g * 16 + j] for j in range(16)]
                gcs = _transpose16(rows)
                vbase = ci * CHUNK + g * 16
                for d in range(C):
                    off = d * VR + vbase
                    a = acc[pl.ds(off, 16)]
                    for c in range(C):
                        a = a + gcs[c] * ws[c][d]
                    acc[pl.ds(off, 16)] = a
            return 0

        lax.fori_loop(0, NCHUNK, c_loop, 0)
        return 0

    lax.fori_loop(0, KVOL, k_loop, 0)

    def t_loop(ci, _):
        for g in range(CHUNK // 16):
            vbase = ci * CHUNK + g * 16
            cols_a = [acc[pl.ds(d * VR + vbase, 16)] for d in range(C)]
            rows_a = _transpose16(cols_a)
            for j in range(16):
                obuf[g * 16 + j] = rows_a[j]
        pltpu.async_copy(obuf,
                         out_hbm.at[pl.ds(tb + ci * CHUNK, CHUNK)],
                         sem).wait()
        return 0

    lax.fori_loop(0, NCHUNK, t_loop, 0)


def _conv(table, nidx, w):
    mesh = plsc.VectorSubcoreMesh(core_axis_name="c", subcore_axis_name="s")
    f = pl.kernel(
        _conv_body,
        mesh=mesh,
        compiler_params=pltpu.CompilerParams(use_tc_tiling_on_sc=False),
        out_type=jax.ShapeDtypeStruct((NPAD, C), jnp.float32),
        scratch_types=[
            pltpu.VMEM((C * VR,), jnp.float32),
            pltpu.VMEM((CHUNK, C), jnp.float32),
            pltpu.VMEM((VR,), jnp.int32),
            pltpu.VMEM((CHUNK, C), jnp.float32),
            pltpu.VMEM_SHARED((KVOL, C, C), jnp.float32),
            pltpu.SMEM((C, C), jnp.float32),
            pltpu.SemaphoreType.DMA,
        ],
    )
    return f(table, nidx, w)


def kernel(features, coors, batch_size, W1, W2):
    n = features.shape[0]
    coors = coors.astype(jnp.int32)
    z, y, x = coors[:, 1], coors[:, 2], coors[:, 3]
    lin = (z * SIDE + y) * SIDE + x
    grid = jnp.full((GRID_SZ,), -1, dtype=jnp.int32)
    grid = grid.at[lin].set(jnp.arange(n, dtype=jnp.int32))

    pad = jnp.full((NPAD - n,), -1000, dtype=jnp.int32)
    zf = jnp.concatenate([z, pad])
    yf = jnp.concatenate([y, pad])
    xf = jnp.concatenate([x, pad])

    feat_ext = jnp.zeros((NPAD, C), jnp.float32).at[:n].set(features)

    nidx = _rulebook(zf, yf, xf, grid)
    out1 = _conv(feat_ext, nidx, W1)
    out2 = _conv(out1, nidx, W2)
    return out2[:n]
